# baseline (speedup 1.0000x reference)
import jax
import jax.numpy as jnp
from jax.experimental import pallas as pl
from jax.experimental.pallas import tpu as pltpu

_EPS = 1e-5
_TOPKS = (192, 256, 288, 307)
_INT32_MIN = -2147483648


def _sortable_int(x):
    i = jax.lax.bitcast_convert_type(x, jnp.int32)
    return jnp.where(i < 0, jnp.int32(_INT32_MIN) - i, i)


def _kth_threshold(keys, k, lo, hi):
    def body(_, carry):
        lo, hi = carry
        mid = (lo & hi) + ((lo ^ hi) >> 1)
        cnt = jnp.sum((keys >= mid).astype(jnp.int32), axis=1, keepdims=True)
        pred = cnt >= k
        return jnp.where(pred, mid, lo), jnp.where(pred, hi, mid)

    lo, hi = jax.lax.fori_loop(0, 32, body, (lo, hi))
    return lo


def _body(x1_ref, x2_ref, ln1w_ref, ln1b_ref, rw_ref, rb_ref, ln2w_ref,
          ln2b_ref, a1_ref, a2_ref, a3_ref, a4_ref, out_ref):
    coef_refs = (a1_ref, a2_ref, a3_ref, a4_ref)
    f32 = jnp.float32
    x1 = x1_ref[0]
    x2 = x2_ref[0]
    w1 = ln1w_ref[...].reshape(1, -1)
    b1 = ln1b_ref[...].reshape(1, -1)

    def ln_rows(x):
        mu = jnp.mean(x, axis=1, keepdims=True)
        xc = x - mu
        var = jnp.mean(xc * xc, axis=1, keepdims=True)
        return xc * jax.lax.rsqrt(var + _EPS) * w1 + b1

    n1 = ln_rows(x1)
    n2 = ln_rows(x2)

    ke = jnp.exp(n1 - jnp.max(n1, axis=0, keepdims=True))
    qe = jnp.exp(n2 - jnp.max(n2, axis=1, keepdims=True))
    ones8N = jnp.ones((8, x1.shape[0]), f32)
    kesum = jax.lax.dot_general(ones8N, ke, (((1,), (0,)), ((), ())),
                                preferred_element_type=f32)[:1]

    ctxT = jax.lax.dot_general(n1, ke, (((0,), (0,)), ((), ())),
                               preferred_element_type=f32) / kesum

    D = ctxT.shape[0]
    keysT = _sortable_int(ctxT)
    m = jnp.max(ctxT, axis=0, keepdims=True)
    eT = jnp.exp(ctxT - m)

    i16 = jnp.int16
    khi = (keysT >> 16).astype(i16)
    klo = ((keysT & 0xFFFF) ^ 0x8000).astype(i16)
    onesb = jnp.ones((8, D), jnp.bfloat16)
    ones8 = jnp.ones((8, D), f32)
    oneb = jnp.bfloat16(1.0)
    zerob = jnp.bfloat16(0.0)
    kvs = tuple(jnp.full((1, D), float(k), f32) for k in _TOPKS)

    def count(maskb):
        return jax.lax.dot_general(onesb, maskb, (((1,), (0,)), ((), ())),
                                   preferred_element_type=f32)[:1]

    los = [jnp.full((1, D), -32768, jnp.int32)] * 4
    his = [jnp.full((1, D), 32768, jnp.int32)] * 4
    for _ in range(16):
        for i in range(4):
            mid = (los[i] + his[i]) >> 1
            maskb = jnp.where(khi >= mid.astype(i16), oneb, zerob)
            pred = count(maskb) >= kvs[i]
            los[i] = jnp.where(pred, mid, los[i])
            his[i] = jnp.where(pred, his[i], mid)

    gcnts, eqbs = [], []
    for i in range(4):
        th = los[i].astype(i16)
        gcnts.append(count(jnp.where(khi > th, oneb, zerob)))
        eqbs.append(jnp.where(khi == th, oneb, zerob))
    los2 = [jnp.zeros((1, D), jnp.int32)] * 4
    his2 = [jnp.full((1, D), 65536, jnp.int32)] * 4
    for _ in range(16):
        for i in range(4):
            mid = (los2[i] + his2[i]) >> 1
            incb = jnp.where(klo >= (mid ^ 0x8000).astype(i16), eqbs[i], zerob)
            pred = (gcnts[i] + count(incb)) >= kvs[i]
            los2[i] = jnp.where(pred, mid, los2[i])
            his2[i] = jnp.where(pred, his2[i], mid)

    coeffT = jnp.zeros_like(ctxT)
    for i in range(4):
        t = (los[i] << 16) | los2[i]
        maskT = jnp.where(keysT >= t, 1.0, 0.0)
        s = jax.lax.dot_general(ones8, eT * maskT, (((1,), (0,)), ((), ())),
                                preferred_element_type=f32)[:1]
        coeffT += maskT * (coef_refs[i][0] / s)

    wmatT = eT * coeffT
    qinv = 1.0 / jax.lax.dot_general(ones8, qe, (((1,), (1,)), ((), ())),
                                     preferred_element_type=f32)[:1]
    att = jax.lax.dot_general(wmatT, qe, (((0,), (1,)), ((), ())),
                              preferred_element_type=f32) * qinv
    rows8 = jnp.concatenate(
        [rb_ref[...].reshape(1, -1), ln2w_ref[...].reshape(1, -1),
         ln2b_ref[...].reshape(1, -1)] + [jnp.zeros((5, 2 * D), f32)], axis=0)
    cols = jax.lax.transpose(rows8, (1, 0))
    rb_c = cols[:, 0:1]
    w2_c = cols[:, 1:2]
    b2_c = cols[:, 2:3]

    outp = jax.lax.dot_general(rw_ref[...], att, (((1,), (0,)), ((), ())),
                               preferred_element_type=f32)
    outp = outp + rb_c

    O = 2 * D
    ones8o = jnp.ones((8, O), f32)
    s1 = jax.lax.dot_general(ones8o, outp, (((1,), (0,)), ((), ())),
                             preferred_element_type=f32)[:1]
    s2 = jax.lax.dot_general(ones8o, outp * outp, (((1,), (0,)), ((), ())),
                             preferred_element_type=f32)[:1]
    mu = s1 * (1.0 / O)
    var = s2 * (1.0 / O) - mu * mu
    rs = jax.lax.rsqrt(var + _EPS)
    out_ref[0] = (outp - mu) * rs * w2_c + b2_c


def _build(B, N, C, interpret=False):
    return pl.pallas_call(
        _body,
        grid=(B,),
        in_specs=[
            pl.BlockSpec((1, N, C), lambda b: (b, 0, 0)),
            pl.BlockSpec((1, N, C), lambda b: (b, 0, 0)),
            pl.BlockSpec((C,), lambda b: (0,)),
            pl.BlockSpec((C,), lambda b: (0,)),
            pl.BlockSpec((2 * C, C), lambda b: (0, 0)),
            pl.BlockSpec((2 * C,), lambda b: (0,)),
            pl.BlockSpec((2 * C,), lambda b: (0,)),
            pl.BlockSpec((2 * C,), lambda b: (0,)),
            pl.BlockSpec(memory_space=pltpu.SMEM),
            pl.BlockSpec(memory_space=pltpu.SMEM),
            pl.BlockSpec(memory_space=pltpu.SMEM),
            pl.BlockSpec(memory_space=pltpu.SMEM),
        ],
        out_specs=pl.BlockSpec((1, 2 * C, N), lambda b: (b, 0, 0)),
        out_shape=jax.ShapeDtypeStruct((B, 2 * C, N), jnp.float32),
        interpret=interpret,
    )


def kernel(x1, x2, ln1_w, ln1_b, reproj_w, reproj_b, ln2_w, ln2_b,
           a1, a2, a3, a4):
    B, H, W, C = x1.shape
    N = H * W
    x1f = x1.reshape(B, N, C)
    x2f = x2.reshape(B, N, C)
    out = _build(B, N, C)(
        x1f, x2f, ln1_w, ln1_b, reproj_w,
        reproj_b, ln2_w, ln2_b, a1, a2, a3, a4)
    return out.reshape(B, 2 * C, H, W)


# speedup vs baseline: 1.0270x; 1.0270x over previous
import jax
import jax.numpy as jnp
from jax.experimental import pallas as pl
from jax.experimental.pallas import tpu as pltpu

_EPS = 1e-5
_TOPKS = (192, 256, 288, 307)
_INT32_MIN = -2147483648


def _sortable_int(x):
    i = jax.lax.bitcast_convert_type(x, jnp.int32)
    return jnp.where(i < 0, jnp.int32(_INT32_MIN) - i, i)


def _kth_threshold(keys, k, lo, hi):
    def body(_, carry):
        lo, hi = carry
        mid = (lo & hi) + ((lo ^ hi) >> 1)
        cnt = jnp.sum((keys >= mid).astype(jnp.int32), axis=1, keepdims=True)
        pred = cnt >= k
        return jnp.where(pred, mid, lo), jnp.where(pred, hi, mid)

    lo, hi = jax.lax.fori_loop(0, 32, body, (lo, hi))
    return lo


def _body(x1_ref, x2_ref, ln1w_ref, ln1b_ref, rw_ref, rb_ref, ln2w_ref,
          ln2b_ref, a1_ref, a2_ref, a3_ref, a4_ref, out_ref):
    coef_refs = (a1_ref, a2_ref, a3_ref, a4_ref)
    f32 = jnp.float32
    x1 = x1_ref[0]
    x2 = x2_ref[0]
    w1 = ln1w_ref[...].reshape(1, -1)
    b1 = ln1b_ref[...].reshape(1, -1)

    def ln_rows(x):
        mu = jnp.mean(x, axis=1, keepdims=True)
        xc = x - mu
        var = jnp.mean(xc * xc, axis=1, keepdims=True)
        return xc * jax.lax.rsqrt(var + _EPS) * w1 + b1

    n1 = ln_rows(x1)
    n2 = ln_rows(x2)

    ke = jnp.exp(n1 - jnp.max(n1, axis=0, keepdims=True))
    ks = ke / jnp.sum(ke, axis=0, keepdims=True)
    qe = jnp.exp(n2 - jnp.max(n2, axis=1, keepdims=True))
    qs = qe / jnp.sum(qe, axis=1, keepdims=True)

    ctxT = jax.lax.dot_general(n1, ks, (((0,), (0,)), ((), ())),
                               preferred_element_type=f32)

    D = ctxT.shape[0]
    keysT = _sortable_int(ctxT)
    m = jnp.max(ctxT, axis=0, keepdims=True)
    eT = jnp.exp(ctxT - m)

    keysT4 = jnp.concatenate([keysT, keysT, keysT, keysT], axis=1)
    lo = jnp.min(keysT4, axis=0, keepdims=True)
    hi = jnp.max(keysT4, axis=0, keepdims=True) + 1
    lane = jax.lax.broadcasted_iota(jnp.int32, (1, 4 * D), 1) // D
    kv = jnp.where(lane == 0, _TOPKS[0],
                   jnp.where(lane == 1, _TOPKS[1],
                             jnp.where(lane == 2, _TOPKS[2], _TOPKS[3])))
    kv = kv.astype(f32)
    ones8 = jnp.ones((8, D), f32)

    for _ in range(32):
        mid = (lo & hi) + ((lo ^ hi) >> 1)
        maskf = jnp.where(keysT4 >= mid, 1.0, 0.0)
        cnt = jax.lax.dot_general(ones8, maskf, (((1,), (0,)), ((), ())),
                                  preferred_element_type=f32)[:1]
        pred = cnt >= kv
        lo = jnp.where(pred, mid, lo)
        hi = jnp.where(pred, hi, mid)

    coeffT = jnp.zeros_like(ctxT)
    for i in range(4):
        t = lo[:, i * D:(i + 1) * D]
        maskT = jnp.where(keysT >= t, 1.0, 0.0)
        s = jax.lax.dot_general(ones8, eT * maskT, (((1,), (0,)), ((), ())),
                                preferred_element_type=f32)[:1]
        coeffT += maskT * (coef_refs[i][0] / s)

    wmatT = eT * coeffT
    att = jax.lax.dot_general(wmatT, qs, (((0,), (1,)), ((), ())),
                              preferred_element_type=f32)
    rows8 = jnp.concatenate(
        [rb_ref[...].reshape(1, -1), ln2w_ref[...].reshape(1, -1),
         ln2b_ref[...].reshape(1, -1)] + [jnp.zeros((5, 2 * D), f32)], axis=0)
    cols = jax.lax.transpose(rows8, (1, 0))
    rb_c = cols[:, 0:1]
    w2_c = cols[:, 1:2]
    b2_c = cols[:, 2:3]

    outp = jax.lax.dot_general(rw_ref[...], att, (((1,), (0,)), ((), ())),
                               preferred_element_type=f32)
    outp = outp + rb_c

    mu = jnp.mean(outp, axis=0, keepdims=True)
    oc = outp - mu
    var = jnp.mean(oc * oc, axis=0, keepdims=True)
    out_ref[0] = oc * jax.lax.rsqrt(var + _EPS) * w2_c + b2_c


def _build(B, N, C):
    return pl.pallas_call(
        _body,
        grid=(B,),
        in_specs=[
            pl.BlockSpec((1, N, C), lambda b: (b, 0, 0)),
            pl.BlockSpec((1, N, C), lambda b: (b, 0, 0)),
            pl.BlockSpec((C,), lambda b: (0,)),
            pl.BlockSpec((C,), lambda b: (0,)),
            pl.BlockSpec((2 * C, C), lambda b: (0, 0)),
            pl.BlockSpec((2 * C,), lambda b: (0,)),
            pl.BlockSpec((2 * C,), lambda b: (0,)),
            pl.BlockSpec((2 * C,), lambda b: (0,)),
            pl.BlockSpec(memory_space=pltpu.SMEM),
            pl.BlockSpec(memory_space=pltpu.SMEM),
            pl.BlockSpec(memory_space=pltpu.SMEM),
            pl.BlockSpec(memory_space=pltpu.SMEM),
        ],
        out_specs=pl.BlockSpec((1, 2 * C, N), lambda b: (b, 0, 0)),
        out_shape=jax.ShapeDtypeStruct((B, 2 * C, N), jnp.float32),
    )


def kernel(x1, x2, ln1_w, ln1_b, reproj_w, reproj_b, ln2_w, ln2_b,
           a1, a2, a3, a4):
    B, H, W, C = x1.shape
    N = H * W
    x1f = x1.reshape(B, N, C)
    x2f = x2.reshape(B, N, C)
    out = _build(B, N, C)(
        x1f, x2f, ln1_w, ln1_b, reproj_w,
        reproj_b, ln2_w, ln2_b, a1, a2, a3, a4)
    return out.reshape(B, 2 * C, H, W)
